# Initial kernel scaffold; baseline (speedup 1.0000x reference)
import jax
import jax.numpy as jnp
from jax.experimental import pallas as pl


def kernel(x, edge_index, edge_weight, batch, W1, b1, W2, b2, Wfc, bfc):
    raise NotImplementedError("write your pallas kernel here")


# speedup vs baseline: 17.5793x; 17.5793x over previous
import functools

import jax
import jax.numpy as jnp
from jax import lax
from jax.experimental import pallas as pl
from jax.experimental.pallas import tpu as pltpu
from jax.experimental.pallas import tpu_sc as plsc

N = 10000
E = 320000
G = 16

NC = 2
NS = 16
NW = NC * NS
C = 80
EPW = E // NW
NCHUNK = EPW // C
NP = 10240
STRIPE = NP // NS
WB = 128

_mesh = functools.partial(
    plsc.VectorSubcoreMesh, core_axis_name="c", subcore_axis_name="s",
    num_cores=NC, num_subcores=NS)

_SC_PARAMS = pltpu.CompilerParams(use_tc_tiling_on_sc=False)


def _zero_vmem_2d(ref, rows, feat):
    def row(i, _):
        for q in range(feat // 16):
            ref[i, pl.ds(q * 16, 16)] = jnp.zeros((16,), jnp.float32)
        return 0
    lax.fori_loop(0, rows, row, 0)


def _deg_body(dst_hbm, ew_hbm, deg_hbm, dstv, ewv, zb, wb, acc):
    c = lax.axis_index("c")
    s = lax.axis_index("s")
    wid = s * NC + c

    pltpu.sync_copy(dst_hbm.at[wid], dstv)
    pltpu.sync_copy(ew_hbm.at[wid], ewv)

    def zrow(i, _):
        zb[pl.ds(i * 16, 16)] = jnp.zeros((16,), jnp.float32)
        return 0
    lax.fori_loop(0, STRIPE // 16, zrow, 0)
    pltpu.sync_copy(zb, acc.at[pl.ds(s * STRIPE, STRIPE)])
    plsc.subcore_barrier()

    def chunk(j, _):
        pltpu.sync_copy(ewv.at[j], acc.at[dstv.at[j]], add=True)
        return 0
    lax.fori_loop(0, NCHUNK, chunk, 0)
    plsc.subcore_barrier()

    pltpu.sync_copy(acc.at[pl.ds(s * STRIPE, STRIPE)], wb)
    pltpu.sync_copy(wb, deg_hbm.at[c, pl.ds(s * STRIPE, STRIPE)])


def _deg_call(dst3, ew3):
    return pl.kernel(
        _deg_body,
        out_type=jax.ShapeDtypeStruct((NC, NP), jnp.float32),
        mesh=_mesh(),
        scratch_types=[
            pltpu.VMEM((NCHUNK, C), jnp.int32),
            pltpu.VMEM((NCHUNK, C), jnp.float32),
            pltpu.VMEM((STRIPE,), jnp.float32),
            pltpu.VMEM((STRIPE,), jnp.float32),
            pltpu.VMEM_SHARED((NP,), jnp.float32),
        ],
        compiler_params=_SC_PARAMS,
    )(dst3, ew3)


def _spmm_body(feat, src_hbm, dst_hbm, ew_hbm, h_hbm, agg_hbm,
               srcv, dstv, ewv, rows, zb, acc, sem):
    c = lax.axis_index("c")
    s = lax.axis_index("s")
    wid = s * NC + c
    nq = feat // 16

    pltpu.sync_copy(src_hbm.at[wid], srcv)
    pltpu.sync_copy(dst_hbm.at[wid], dstv)
    pltpu.sync_copy(ew_hbm.at[wid], ewv)

    _zero_vmem_2d(zb, WB, feat)
    for k in range(STRIPE // WB):
        pltpu.sync_copy(zb, acc.at[pl.ds(s * STRIPE + k * WB, WB), :])
    plsc.subcore_barrier()

    lane_idx = [jnp.full((16,), l, jnp.int32) for l in range(16)]

    def chunk(j, _):
        pltpu.async_copy(h_hbm.at[srcv.at[j]], rows, sem).wait()

        def group(g, _):
            wv = ewv[j, pl.ds(g * 16, 16)]
            for l in range(16):
                wl = lax.gather(
                    wv, lane_idx[l][:, None],
                    lax.GatherDimensionNumbers(
                        offset_dims=(), collapsed_slice_dims=(0,),
                        start_index_map=(0,)),
                    (1,), mode=lax.GatherScatterMode.PROMISE_IN_BOUNDS)
                e = g * 16 + l
                for q in range(nq):
                    rows[e, pl.ds(q * 16, 16)] = (
                        rows[e, pl.ds(q * 16, 16)] * wl)
            return 0
        lax.fori_loop(0, C // 16, group, 0)

        pltpu.sync_copy(rows, acc.at[dstv.at[j]], add=True)
        return 0
    lax.fori_loop(0, NCHUNK, chunk, 0)
    plsc.subcore_barrier()

    for k in range(STRIPE // WB):
        pltpu.sync_copy(acc.at[pl.ds(s * STRIPE + k * WB, WB), :], zb)
        pltpu.sync_copy(zb, agg_hbm.at[c, pl.ds(s * STRIPE + k * WB, WB), :])


def _spmm_call(src3, dst3, ew3, h, feat):
    return pl.kernel(
        functools.partial(_spmm_body, feat),
        out_type=jax.ShapeDtypeStruct((NC, NP, feat), jnp.float32),
        mesh=_mesh(),
        scratch_types=[
            pltpu.VMEM((NCHUNK, C), jnp.int32),
            pltpu.VMEM((NCHUNK, C), jnp.int32),
            pltpu.VMEM((NCHUNK, C), jnp.float32),
            pltpu.VMEM((C, feat), jnp.float32),
            pltpu.VMEM((WB, feat), jnp.float32),
            pltpu.VMEM_SHARED((NP, feat), jnp.float32),
            pltpu.SemaphoreType.DMA,
        ],
        compiler_params=_SC_PARAMS,
    )(src3, dst3, ew3, h)


def _k1_body(x_ref, w1_ref, degt_ref, h1p_ref, dis_ref):
    deg = degt_ref[:, 0:1] + degt_ref[:, 1:2] + 1.0
    dis = lax.rsqrt(deg)
    dis_ref[...] = dis
    h = jnp.dot(x_ref[...], w1_ref[...], preferred_element_type=jnp.float32)
    h1p_ref[...] = dis * h


def _k1_call(x, W1, degT):
    blk = 1000
    grid = N // blk
    return pl.pallas_call(
        _k1_body,
        grid=(grid,),
        in_specs=[
            pl.BlockSpec((blk, 128), lambda i: (i, 0)),
            pl.BlockSpec((128, 64), lambda i: (0, 0)),
            pl.BlockSpec((blk, NC), lambda i: (i, 0)),
        ],
        out_specs=[
            pl.BlockSpec((blk, 64), lambda i: (i, 0)),
            pl.BlockSpec((blk, 1), lambda i: (i, 0)),
        ],
        out_shape=[
            jax.ShapeDtypeStruct((N, 64), jnp.float32),
            jax.ShapeDtypeStruct((N, 1), jnp.float32),
        ],
    )(x, W1, degT)


def _k2_body(a0_ref, a1_ref, h1p_ref, dis_ref, b1_ref, w2_ref, h2p_ref):
    dis = dis_ref[...]
    z = dis * (a0_ref[...] + a1_ref[...] + h1p_ref[...]) + b1_ref[...]
    z = jnp.maximum(z, 0.0)
    h2 = jnp.dot(z, w2_ref[...], preferred_element_type=jnp.float32)
    h2p_ref[...] = dis * h2


def _k2_call(a0, a1, h1p, dis, b1, W2):
    blk = 1000
    grid = N // blk
    return pl.pallas_call(
        _k2_body,
        grid=(grid,),
        in_specs=[
            pl.BlockSpec((blk, 64), lambda i: (i, 0)),
            pl.BlockSpec((blk, 64), lambda i: (i, 0)),
            pl.BlockSpec((blk, 64), lambda i: (i, 0)),
            pl.BlockSpec((blk, 1), lambda i: (i, 0)),
            pl.BlockSpec((1, 64), lambda i: (0, 0)),
            pl.BlockSpec((64, 32), lambda i: (0, 0)),
        ],
        out_specs=pl.BlockSpec((blk, 32), lambda i: (i, 0)),
        out_shape=jax.ShapeDtypeStruct((N, 32), jnp.float32),
    )(a0, a1, h1p, dis, b1, W2)


def _k3_body(a0_ref, a1_ref, h2p_ref, dis_ref, b2_ref, batch_ref, wfc_ref,
             bfc_ref, out_ref):
    dis = dis_ref[...]
    z = dis * (a0_ref[...] + a1_ref[...] + h2p_ref[...]) + b2_ref[...]
    z = jnp.maximum(z, 0.0)
    gids = lax.broadcasted_iota(jnp.int32, (G, N), 0)
    onehot = (gids == batch_ref[...]).astype(jnp.float32)
    sums = jnp.dot(onehot, z, preferred_element_type=jnp.float32)
    cnt = jnp.sum(onehot, axis=1, keepdims=True)
    pooled = sums / jnp.maximum(cnt, 1.0)
    out_ref[...] = jnp.dot(pooled, wfc_ref[...],
                           preferred_element_type=jnp.float32) + bfc_ref[...]


def _k3_call(a0, a1, h2p, dis, b2, batch2d, Wfc, bfc, flat):
    return pl.pallas_call(
        _k3_body,
        out_shape=jax.ShapeDtypeStruct((G, flat), jnp.float32),
    )(a0, a1, h2p, dis, b2, batch2d, Wfc, bfc)


def kernel(x, edge_index, edge_weight, batch, W1, b1, W2, b2, Wfc, bfc):
    flat = Wfc.shape[1]
    src3 = edge_index[0].reshape(NW, NCHUNK, C)
    dst3 = edge_index[1].reshape(NW, NCHUNK, C)
    ew3 = edge_weight.reshape(NW, NCHUNK, C)

    degp = _deg_call(dst3, ew3)
    degT = degp.T
    h1p, dis = _k1_call(x, W1, degT)

    agg1 = _spmm_call(src3, dst3, ew3, h1p, 64)
    h2p = _k2_call(agg1[0, :N], agg1[1, :N], h1p, dis, b1.reshape(1, 64), W2)

    agg2 = _spmm_call(src3, dst3, ew3, h2p, 32)
    return _k3_call(agg2[0, :N], agg2[1, :N], h2p, dis, b2.reshape(1, 32),
                    batch.reshape(1, N), Wfc, bfc.reshape(1, flat), flat)


# speedup vs baseline: 25.8954x; 1.4731x over previous
import functools

import jax
import jax.numpy as jnp
from jax import lax
from jax.experimental import pallas as pl
from jax.experimental.pallas import tpu as pltpu
from jax.experimental.pallas import tpu_sc as plsc

N = 10000
E = 320000
G = 16

NC = 2
NS = 16
NW = NC * NS
C = 80
EPW = E // NW
NCHUNK = EPW // C
NP = 10240
STRIPE = NP // NS
WB = 128

_mesh = functools.partial(
    plsc.VectorSubcoreMesh, core_axis_name="c", subcore_axis_name="s",
    num_cores=NC, num_subcores=NS)

_SC_PARAMS = pltpu.CompilerParams(use_tc_tiling_on_sc=False)


def _zero_vmem_2d(ref, rows, feat):
    def row(i, _):
        for q in range(feat // 16):
            ref[i, pl.ds(q * 16, 16)] = jnp.zeros((16,), jnp.float32)
        return 0
    lax.fori_loop(0, rows, row, 0)


def _deg_body(dst_hbm, ew_hbm, deg_hbm, dstv, ewv, zb, wb, acc):
    c = lax.axis_index("c")
    s = lax.axis_index("s")
    wid = s * NC + c

    pltpu.sync_copy(dst_hbm.at[wid], dstv)
    pltpu.sync_copy(ew_hbm.at[wid], ewv)

    def zrow(i, _):
        zb[pl.ds(i * 16, 16)] = jnp.zeros((16,), jnp.float32)
        return 0
    lax.fori_loop(0, STRIPE // 16, zrow, 0)
    pltpu.sync_copy(zb, acc.at[pl.ds(s * STRIPE, STRIPE)])
    plsc.subcore_barrier()

    def chunk(j, _):
        pltpu.sync_copy(ewv.at[j], acc.at[dstv.at[j]], add=True)
        return 0
    lax.fori_loop(0, NCHUNK, chunk, 0)
    plsc.subcore_barrier()

    pltpu.sync_copy(acc.at[pl.ds(s * STRIPE, STRIPE)], wb)
    pltpu.sync_copy(wb, deg_hbm.at[c, pl.ds(s * STRIPE, STRIPE)])


def _deg_call(dst3, ew3):
    return pl.kernel(
        _deg_body,
        out_type=jax.ShapeDtypeStruct((NC, NP), jnp.float32),
        mesh=_mesh(),
        scratch_types=[
            pltpu.VMEM((NCHUNK, C), jnp.int32),
            pltpu.VMEM((NCHUNK, C), jnp.float32),
            pltpu.VMEM((STRIPE,), jnp.float32),
            pltpu.VMEM((STRIPE,), jnp.float32),
            pltpu.VMEM_SHARED((NP,), jnp.float32),
        ],
        compiler_params=_SC_PARAMS,
    )(dst3, ew3)


def _spmm_body(feat, src_hbm, dst_hbm, ew_hbm, h_hbm, agg_hbm,
               srcv, dstv, ewv, rows0, rows1, zb, hsp, acc, gsem0, gsem1):
    c = lax.axis_index("c")
    s = lax.axis_index("s")
    wid = s * NC + c
    nq = feat // 16

    pltpu.sync_copy(src_hbm.at[wid], srcv)
    pltpu.sync_copy(dst_hbm.at[wid], dstv)
    pltpu.sync_copy(ew_hbm.at[wid], ewv)

    HB = N // NS
    pltpu.sync_copy(h_hbm.at[pl.ds(s * HB, HB), :],
                    hsp.at[pl.ds(s * HB, HB), :])

    _zero_vmem_2d(zb, WB, feat)
    for k in range(STRIPE // WB):
        pltpu.sync_copy(zb, acc.at[pl.ds(s * STRIPE + k * WB, WB), :])
    plsc.subcore_barrier()

    lane_idx = [jnp.full((16,), l, jnp.int32) for l in range(16)]
    bcast_dnums = lax.GatherDimensionNumbers(
        offset_dims=(), collapsed_slice_dims=(0,), start_index_map=(0,))

    def scale(buf, j):
        def group(g, _):
            wv = ewv[j, pl.ds(g * 16, 16)]
            for l in range(16):
                wl = lax.gather(wv, lane_idx[l][:, None], bcast_dnums, (1,),
                                mode=lax.GatherScatterMode.PROMISE_IN_BOUNDS)
                e = g * 16 + l
                for q in range(nq):
                    buf[e, pl.ds(q * 16, 16)] = (
                        buf[e, pl.ds(q * 16, 16)] * wl)
            return 0
        lax.fori_loop(0, C // 16, group, 0)

    def start_g(j, buf, sem):
        pltpu.async_copy(hsp.at[srcv.at[j]], buf, sem)

    def wait_g(j, buf, sem):
        pltpu.make_async_copy(hsp.at[srcv.at[j]], buf, sem).wait()

    start_g(0, rows0, gsem0)

    def pair(i, _):
        j0 = 2 * i
        start_g(j0 + 1, rows1, gsem1)
        wait_g(j0, rows0, gsem0)
        scale(rows0, j0)
        pltpu.sync_copy(rows0, acc.at[dstv.at[j0]], add=True)
        start_g(j0 + 2, rows0, gsem0)
        wait_g(j0 + 1, rows1, gsem1)
        scale(rows1, j0 + 1)
        pltpu.sync_copy(rows1, acc.at[dstv.at[j0 + 1]], add=True)
        return 0
    lax.fori_loop(0, (NCHUNK - 1) // 2, pair, 0)

    last = NCHUNK - 1
    wait_g(last, rows0, gsem0)
    scale(rows0, last)
    pltpu.sync_copy(rows0, acc.at[dstv.at[last]], add=True)
    plsc.subcore_barrier()

    for k in range(STRIPE // WB):
        pltpu.sync_copy(acc.at[pl.ds(s * STRIPE + k * WB, WB), :], zb)
        pltpu.sync_copy(zb, agg_hbm.at[c, pl.ds(s * STRIPE + k * WB, WB), :])


def _spmm_call(src3, dst3, ew3, h, feat):
    return pl.kernel(
        functools.partial(_spmm_body, feat),
        out_type=jax.ShapeDtypeStruct((NC, NP, feat), jnp.float32),
        mesh=_mesh(),
        scratch_types=[
            pltpu.VMEM((NCHUNK, C), jnp.int32),
            pltpu.VMEM((NCHUNK, C), jnp.int32),
            pltpu.VMEM((NCHUNK, C), jnp.float32),
            pltpu.VMEM((C, feat), jnp.float32),
            pltpu.VMEM((C, feat), jnp.float32),
            pltpu.VMEM((WB, feat), jnp.float32),
            pltpu.VMEM_SHARED((N, feat), jnp.float32),
            pltpu.VMEM_SHARED((NP, feat), jnp.float32),
            pltpu.SemaphoreType.DMA,
            pltpu.SemaphoreType.DMA,
        ],
        compiler_params=_SC_PARAMS,
    )(src3, dst3, ew3, h)


def _k1_body(x_ref, w1_ref, degt_ref, h1p_ref, dis_ref):
    deg = degt_ref[:, 0:1] + degt_ref[:, 1:2] + 1.0
    dis = lax.rsqrt(deg)
    dis_ref[...] = dis
    h = jnp.dot(x_ref[...], w1_ref[...], preferred_element_type=jnp.float32)
    h1p_ref[...] = dis * h


def _k1_call(x, W1, degT):
    blk = 1000
    grid = N // blk
    return pl.pallas_call(
        _k1_body,
        grid=(grid,),
        in_specs=[
            pl.BlockSpec((blk, 128), lambda i: (i, 0)),
            pl.BlockSpec((128, 64), lambda i: (0, 0)),
            pl.BlockSpec((blk, NC), lambda i: (i, 0)),
        ],
        out_specs=[
            pl.BlockSpec((blk, 64), lambda i: (i, 0)),
            pl.BlockSpec((blk, 1), lambda i: (i, 0)),
        ],
        out_shape=[
            jax.ShapeDtypeStruct((N, 64), jnp.float32),
            jax.ShapeDtypeStruct((N, 1), jnp.float32),
        ],
    )(x, W1, degT)


def _k2_body(a0_ref, a1_ref, h1p_ref, dis_ref, b1_ref, w2_ref, h2p_ref):
    dis = dis_ref[...]
    z = dis * (a0_ref[...] + a1_ref[...] + h1p_ref[...]) + b1_ref[...]
    z = jnp.maximum(z, 0.0)
    h2 = jnp.dot(z, w2_ref[...], preferred_element_type=jnp.float32)
    h2p_ref[...] = dis * h2


def _k2_call(a0, a1, h1p, dis, b1, W2):
    blk = 1000
    grid = N // blk
    return pl.pallas_call(
        _k2_body,
        grid=(grid,),
        in_specs=[
            pl.BlockSpec((blk, 64), lambda i: (i, 0)),
            pl.BlockSpec((blk, 64), lambda i: (i, 0)),
            pl.BlockSpec((blk, 64), lambda i: (i, 0)),
            pl.BlockSpec((blk, 1), lambda i: (i, 0)),
            pl.BlockSpec((1, 64), lambda i: (0, 0)),
            pl.BlockSpec((64, 32), lambda i: (0, 0)),
        ],
        out_specs=pl.BlockSpec((blk, 32), lambda i: (i, 0)),
        out_shape=jax.ShapeDtypeStruct((N, 32), jnp.float32),
    )(a0, a1, h1p, dis, b1, W2)


def _k3_body(a0_ref, a1_ref, h2p_ref, dis_ref, b2_ref, batch_ref, wfc_ref,
             bfc_ref, out_ref):
    dis = dis_ref[...]
    z = dis * (a0_ref[...] + a1_ref[...] + h2p_ref[...]) + b2_ref[...]
    z = jnp.maximum(z, 0.0)
    gids = lax.broadcasted_iota(jnp.int32, (G, N), 0)
    onehot = (gids == batch_ref[...]).astype(jnp.float32)
    sums = jnp.dot(onehot, z, preferred_element_type=jnp.float32)
    cnt = jnp.sum(onehot, axis=1, keepdims=True)
    pooled = sums / jnp.maximum(cnt, 1.0)
    out_ref[...] = jnp.dot(pooled, wfc_ref[...],
                           preferred_element_type=jnp.float32) + bfc_ref[...]


def _k3_call(a0, a1, h2p, dis, b2, batch2d, Wfc, bfc, flat):
    return pl.pallas_call(
        _k3_body,
        out_shape=jax.ShapeDtypeStruct((G, flat), jnp.float32),
    )(a0, a1, h2p, dis, b2, batch2d, Wfc, bfc)


def kernel(x, edge_index, edge_weight, batch, W1, b1, W2, b2, Wfc, bfc):
    flat = Wfc.shape[1]
    src3 = edge_index[0].reshape(NW, NCHUNK, C)
    dst3 = edge_index[1].reshape(NW, NCHUNK, C)
    ew3 = edge_weight.reshape(NW, NCHUNK, C)

    degp = _deg_call(dst3, ew3)
    degT = degp.T
    h1p, dis = _k1_call(x, W1, degT)

    agg1 = _spmm_call(src3, dst3, ew3, h1p, 64)
    h2p = _k2_call(agg1[0, :N], agg1[1, :N], h1p, dis, b1.reshape(1, 64), W2)

    agg2 = _spmm_call(src3, dst3, ew3, h2p, 32)
    return _k3_call(agg2[0, :N], agg2[1, :N], h2p, dis, b2.reshape(1, 32),
                    batch.reshape(1, N), Wfc, bfc.reshape(1, flat), flat)


# speedup vs baseline: 37.9019x; 1.4637x over previous
import functools

import jax
import jax.numpy as jnp
from jax import lax
from jax.experimental import pallas as pl
from jax.experimental.pallas import tpu as pltpu
from jax.experimental.pallas import tpu_sc as plsc

N = 10000
E = 320000
G = 16

NC = 2
NS = 16
NW = NC * NS
C = 64
NCHUNK = 160
EPW = NCHUNK * C
EPAD = NW * EPW
NP = 10240
STRIPE = NP // NS
WB = 128

_mesh = functools.partial(
    plsc.VectorSubcoreMesh, core_axis_name="c", subcore_axis_name="s",
    num_cores=NC, num_subcores=NS)

_SC_PARAMS = pltpu.CompilerParams(use_tc_tiling_on_sc=False)


def _zero_vmem_2d(ref, rows, feat):
    def row(i, _):
        for q in range(feat // 16):
            ref[i, pl.ds(q * 16, 16)] = jnp.zeros((16,), jnp.float32)
        return 0
    lax.fori_loop(0, rows, row, 0)


def _deg_body(dst_hbm, ew_hbm, deg_hbm, dstv, ewv, zb, acc):
    c = lax.axis_index("c")
    s = lax.axis_index("s")
    wid = s * NC + c

    pltpu.sync_copy(dst_hbm.at[wid], dstv)
    pltpu.sync_copy(ew_hbm.at[wid], ewv)

    def zrow(i, _):
        zb[pl.ds(i * 16, 16)] = jnp.zeros((16,), jnp.float32)
        return 0
    lax.fori_loop(0, STRIPE // 16, zrow, 0)
    pltpu.sync_copy(zb, acc.at[pl.ds(s * STRIPE, STRIPE)])
    plsc.subcore_barrier()

    def chunk(j, _):
        pltpu.sync_copy(ewv.at[j], acc.at[dstv.at[j]], add=True)
        return 0
    lax.fori_loop(0, NCHUNK, chunk, 0)
    plsc.subcore_barrier()

    pltpu.sync_copy(acc.at[pl.ds(s * STRIPE, STRIPE)], zb)
    pltpu.sync_copy(zb, deg_hbm.at[c, pl.ds(s * STRIPE, STRIPE)])


def _deg_call(dst3, ew3):
    return pl.kernel(
        _deg_body,
        out_type=jax.ShapeDtypeStruct((NC, NP), jnp.float32),
        mesh=_mesh(),
        scratch_types=[
            pltpu.VMEM((NCHUNK, C), jnp.int32),
            pltpu.VMEM((NCHUNK, C), jnp.float32),
            pltpu.VMEM((STRIPE,), jnp.float32),
            pltpu.VMEM_SHARED((NP,), jnp.float32),
        ],
        compiler_params=_SC_PARAMS,
        name="degkernel",
    )(dst3, ew3)


def _spmm_body(feat, stage, src_hbm, dst_hbm, ew_hbm, h_hbm, agg_hbm,
               srcv, dstv, ewv, rows0, rows1, srows0, srows1, *rest):
    if stage:
        hsp, acc, gsem0, gsem1, ssem0, ssem1 = rest
    else:
        acc, gsem0, gsem1, ssem0, ssem1 = rest
    c = lax.axis_index("c")
    s = lax.axis_index("s")
    wid = s * NC + c
    nq = feat // 16

    pltpu.sync_copy(src_hbm.at[wid], srcv)
    pltpu.sync_copy(dst_hbm.at[wid], dstv)
    pltpu.sync_copy(ew_hbm.at[wid], ewv)

    HB = N // NS
    if stage:
        pltpu.sync_copy(h_hbm.at[pl.ds(s * HB, HB), :],
                        hsp.at[pl.ds(s * HB, HB), :])
        gsrc = hsp
    else:
        gsrc = h_hbm

    _zero_vmem_2d(srows0, C, feat)
    nfull, rem = HB // C, HB % C
    for k in range(nfull):
        pltpu.sync_copy(srows0, acc.at[pl.ds(s * HB + k * C, C), :])
    if rem:
        pltpu.sync_copy(srows0.at[pl.ds(0, rem), :],
                        acc.at[pl.ds(s * HB + nfull * C, rem), :])
    plsc.subcore_barrier()

    _zero_vmem_2d(srows1, C, feat)
    pltpu.async_copy(srows0, acc.at[dstv.at[0]], ssem0, add=True)
    pltpu.async_copy(srows1, acc.at[dstv.at[0]], ssem1, add=True)

    lane_idx = [jnp.full((16,), l, jnp.int32) for l in range(16)]
    bcast_dnums = lax.GatherDimensionNumbers(
        offset_dims=(), collapsed_slice_dims=(0,), start_index_map=(0,))

    def scale(gbuf, sbuf, j):
        def group(g, _):
            wv = ewv[j, pl.ds(g * 16, 16)]
            for l in range(16):
                wl = lax.gather(wv, lane_idx[l][:, None], bcast_dnums, (1,),
                                mode=lax.GatherScatterMode.PROMISE_IN_BOUNDS)
                e = g * 16 + l
                for q in range(nq):
                    sbuf[e, pl.ds(q * 16, 16)] = (
                        gbuf[e, pl.ds(q * 16, 16)] * wl)
            return 0
        lax.fori_loop(0, C // 16, group, 0)

    def start_g(j, buf, sem):
        pltpu.async_copy(gsrc.at[srcv.at[j]], buf, sem)

    def wait_g(j, buf, sem):
        pltpu.make_async_copy(gsrc.at[srcv.at[j]], buf, sem).wait()

    def wait_s(sbuf, sem):
        pltpu.make_async_copy(sbuf, acc.at[dstv.at[0]], sem).wait()

    def step(j, gbuf, gsem, sbuf, ssem, nxt):
        wait_g(j, gbuf, gsem)
        wait_s(sbuf, ssem)
        scale(gbuf, sbuf, j)
        pltpu.async_copy(sbuf, acc.at[dstv.at[j]], ssem, add=True)
        if nxt:
            start_g(j + 2, gbuf, gsem)

    start_g(0, rows0, gsem0)
    start_g(1, rows1, gsem1)

    def pair(i, _):
        j0 = 2 * i
        step(j0, rows0, gsem0, srows0, ssem0, True)
        step(j0 + 1, rows1, gsem1, srows1, ssem1, True)
        return 0
    lax.fori_loop(0, NCHUNK // 2 - 1, pair, 0)
    step(NCHUNK - 2, rows0, gsem0, srows0, ssem0, False)
    step(NCHUNK - 1, rows1, gsem1, srows1, ssem1, False)
    wait_s(srows0, ssem0)
    wait_s(srows1, ssem1)
    plsc.subcore_barrier()

    for k in range(nfull):
        buf = srows0 if k % 2 == 0 else srows1
        pltpu.sync_copy(acc.at[pl.ds(s * HB + k * C, C), :], buf)
        pltpu.sync_copy(buf, agg_hbm.at[c, pl.ds(s * HB + k * C, C), :])
    if rem:
        pltpu.sync_copy(acc.at[pl.ds(s * HB + nfull * C, rem), :],
                        srows0.at[pl.ds(0, rem), :])
        pltpu.sync_copy(srows0.at[pl.ds(0, rem), :],
                        agg_hbm.at[c, pl.ds(s * HB + nfull * C, rem), :])


def _spmm_call(src3, dst3, ew3, h, feat, stage):
    scratch = [
        pltpu.VMEM((NCHUNK, C), jnp.int32),
        pltpu.VMEM((NCHUNK, C), jnp.int32),
        pltpu.VMEM((NCHUNK, C), jnp.float32),
        pltpu.VMEM((C, feat), jnp.float32),
        pltpu.VMEM((C, feat), jnp.float32),
        pltpu.VMEM((C, feat), jnp.float32),
        pltpu.VMEM((C, feat), jnp.float32),
    ]
    if stage:
        scratch.append(pltpu.VMEM_SHARED((N, feat), jnp.float32))
    scratch += [
        pltpu.VMEM_SHARED((N, feat), jnp.float32),
        pltpu.SemaphoreType.DMA,
        pltpu.SemaphoreType.DMA,
        pltpu.SemaphoreType.DMA,
        pltpu.SemaphoreType.DMA,
    ]
    return pl.kernel(
        functools.partial(_spmm_body, feat, stage),
        out_type=jax.ShapeDtypeStruct((NC, N, feat), jnp.float32),
        mesh=_mesh(),
        scratch_types=scratch,
        compiler_params=_SC_PARAMS,
        name=f"spmm{feat}",
    )(src3, dst3, ew3, h)


def _k1_body(x_ref, w1_ref, degt_ref, h1p_ref, dis_ref):
    deg = degt_ref[:, 0:1] + degt_ref[:, 1:2] + 1.0
    dis = lax.rsqrt(deg)
    dis_ref[...] = dis
    h = jnp.dot(x_ref[...], w1_ref[...], preferred_element_type=jnp.float32)
    h1p_ref[...] = dis * h


def _k1_call(x, W1, degT):
    blk = 1000
    grid = N // blk
    return pl.pallas_call(
        _k1_body,
        grid=(grid,),
        in_specs=[
            pl.BlockSpec((blk, 128), lambda i: (i, 0)),
            pl.BlockSpec((128, 64), lambda i: (0, 0)),
            pl.BlockSpec((blk, NC), lambda i: (i, 0)),
        ],
        out_specs=[
            pl.BlockSpec((blk, 64), lambda i: (i, 0)),
            pl.BlockSpec((blk, 1), lambda i: (i, 0)),
        ],
        out_shape=[
            jax.ShapeDtypeStruct((N, 64), jnp.float32),
            jax.ShapeDtypeStruct((N, 1), jnp.float32),
        ],
    )(x, W1, degT)


def _k2_body(a0_ref, a1_ref, h1p_ref, dis_ref, b1_ref, w2_ref, h2p_ref):
    dis = dis_ref[...]
    z = dis * (a0_ref[...] + a1_ref[...] + h1p_ref[...]) + b1_ref[...]
    z = jnp.maximum(z, 0.0)
    h2 = jnp.dot(z, w2_ref[...], preferred_element_type=jnp.float32)
    h2p_ref[...] = dis * h2


def _k2_call(a0, a1, h1p, dis, b1, W2):
    blk = 1000
    grid = N // blk
    return pl.pallas_call(
        _k2_body,
        grid=(grid,),
        in_specs=[
            pl.BlockSpec((blk, 64), lambda i: (i, 0)),
            pl.BlockSpec((blk, 64), lambda i: (i, 0)),
            pl.BlockSpec((blk, 64), lambda i: (i, 0)),
            pl.BlockSpec((blk, 1), lambda i: (i, 0)),
            pl.BlockSpec((1, 64), lambda i: (0, 0)),
            pl.BlockSpec((64, 32), lambda i: (0, 0)),
        ],
        out_specs=pl.BlockSpec((blk, 32), lambda i: (i, 0)),
        out_shape=jax.ShapeDtypeStruct((N, 32), jnp.float32),
    )(a0, a1, h1p, dis, b1, W2)


def _k3_body(a0_ref, a1_ref, h2p_ref, dis_ref, b2_ref, batch_ref, wfc_ref,
             bfc_ref, out_ref):
    dis = dis_ref[...]
    z = dis * (a0_ref[...] + a1_ref[...] + h2p_ref[...]) + b2_ref[...]
    z = jnp.maximum(z, 0.0)
    gids = lax.broadcasted_iota(jnp.int32, (G, N), 0)
    onehot = (gids == batch_ref[...]).astype(jnp.float32)
    sums = jnp.dot(onehot, z, preferred_element_type=jnp.float32)
    cnt = jnp.sum(onehot, axis=1, keepdims=True)
    pooled = sums / jnp.maximum(cnt, 1.0)
    out_ref[...] = jnp.dot(pooled, wfc_ref[...],
                           preferred_element_type=jnp.float32) + bfc_ref[...]


def _k3_call(a0, a1, h2p, dis, b2, batch2d, Wfc, bfc, flat):
    return pl.pallas_call(
        _k3_body,
        out_shape=jax.ShapeDtypeStruct((G, flat), jnp.float32),
    )(a0, a1, h2p, dis, b2, batch2d, Wfc, bfc)


def kernel(x, edge_index, edge_weight, batch, W1, b1, W2, b2, Wfc, bfc):
    flat = Wfc.shape[1]
    pad = EPAD - E
    zi = jnp.zeros((pad,), jnp.int32)
    src3 = jnp.concatenate([edge_index[0], zi]).reshape(NW, NCHUNK, C)
    dst3 = jnp.concatenate([edge_index[1], zi]).reshape(NW, NCHUNK, C)
    ew3 = jnp.concatenate(
        [edge_weight, jnp.zeros((pad,), jnp.float32)]).reshape(NW, NCHUNK, C)

    degp = _deg_call(dst3, ew3)
    degT = degp.T
    h1p, dis = _k1_call(x, W1, degT)

    agg1 = _spmm_call(src3, dst3, ew3, h1p, 64, True)
    h2p = _k2_call(agg1[0], agg1[1], h1p, dis, b1.reshape(1, 64), W2)

    agg2 = _spmm_call(src3, dst3, ew3, h2p, 32, True)
    return _k3_call(agg2[0, :N], agg2[1, :N], h2p, dis, b2.reshape(1, 32),
                    batch.reshape(1, N), Wfc, bfc.reshape(1, flat), flat)


# speedup vs baseline: 39.5660x; 1.0439x over previous
import functools

import jax
import jax.numpy as jnp
from jax import lax
from jax.experimental import pallas as pl
from jax.experimental.pallas import tpu as pltpu
from jax.experimental.pallas import tpu_sc as plsc

N = 10000
E = 320000
G = 16

NC = 2
NS = 16
NW = NC * NS
C = 64
NCHUNK = 160
EPW = NCHUNK * C
EPAD = NW * EPW
NP = 10240
STRIPE = NP // NS
WB = 128

_mesh = functools.partial(
    plsc.VectorSubcoreMesh, core_axis_name="c", subcore_axis_name="s",
    num_cores=NC, num_subcores=NS)

_SC_PARAMS = pltpu.CompilerParams(use_tc_tiling_on_sc=False)


def _zero_vmem_2d(ref, rows, feat):
    def row(i, _):
        for q in range(feat // 16):
            ref[i, pl.ds(q * 16, 16)] = jnp.zeros((16,), jnp.float32)
        return 0
    lax.fori_loop(0, rows, row, 0)


def _deg_body(dst_hbm, ew_hbm, deg_hbm, dstv, ewv, zb, acc):
    c = lax.axis_index("c")
    s = lax.axis_index("s")
    wid = s * NC + c

    pltpu.sync_copy(dst_hbm.at[wid], dstv)
    pltpu.sync_copy(ew_hbm.at[wid], ewv)

    def zrow(i, _):
        zb[pl.ds(i * 16, 16)] = jnp.zeros((16,), jnp.float32)
        return 0
    lax.fori_loop(0, STRIPE // 16, zrow, 0)
    pltpu.sync_copy(zb, acc.at[pl.ds(s * STRIPE, STRIPE)])
    plsc.subcore_barrier()

    def chunk(j, _):
        pltpu.sync_copy(ewv.at[j], acc.at[dstv.at[j]], add=True)
        return 0
    lax.fori_loop(0, EPW // 128, chunk, 0)
    plsc.subcore_barrier()

    pltpu.sync_copy(acc.at[pl.ds(s * STRIPE, STRIPE)], zb)
    pltpu.sync_copy(zb, deg_hbm.at[c, pl.ds(s * STRIPE, STRIPE)])


def _deg_call(dst2, ew2):
    return pl.kernel(
        _deg_body,
        out_type=jax.ShapeDtypeStruct((NC, NP), jnp.float32),
        mesh=_mesh(),
        scratch_types=[
            pltpu.VMEM((EPW // 128, 128), jnp.int32),
            pltpu.VMEM((EPW // 128, 128), jnp.float32),
            pltpu.VMEM((STRIPE,), jnp.float32),
            pltpu.VMEM_SHARED((NP,), jnp.float32),
        ],
        compiler_params=_SC_PARAMS,
        name="degkernel",
    )(dst2, ew2)


def _spmm_body(feat, stage, src_hbm, dst_hbm, ew_hbm, h_hbm, agg_hbm,
               srcv, dstv, ewv, rows0, rows1, srows0, srows1, *rest):
    if stage:
        hsp, acc, gsem0, gsem1, ssem0, ssem1 = rest
    else:
        acc, gsem0, gsem1, ssem0, ssem1 = rest
    c = lax.axis_index("c")
    s = lax.axis_index("s")
    wid = s * NC + c
    nq = feat // 16

    pltpu.async_copy(src_hbm.at[wid], srcv, gsem0)
    pltpu.async_copy(dst_hbm.at[wid], dstv, gsem0)
    pltpu.async_copy(ew_hbm.at[wid], ewv, gsem0)

    HB = N // NS
    if stage:
        pltpu.async_copy(h_hbm.at[pl.ds(s * HB, HB), :],
                         hsp.at[pl.ds(s * HB, HB), :], gsem1)
        gsrc = hsp
    else:
        gsrc = h_hbm

    _zero_vmem_2d(srows0, C, feat)
    nfull, rem = HB // C, HB % C
    for k in range(nfull):
        pltpu.async_copy(srows0, acc.at[pl.ds(s * HB + k * C, C), :], ssem0)
    if rem:
        pltpu.async_copy(srows0.at[pl.ds(0, rem), :],
                         acc.at[pl.ds(s * HB + nfull * C, rem), :], ssem0)

    pltpu.make_async_copy(src_hbm.at[wid], srcv, gsem0).wait()
    pltpu.make_async_copy(src_hbm.at[wid], dstv, gsem0).wait()
    pltpu.make_async_copy(ew_hbm.at[wid], ewv, gsem0).wait()
    if stage:
        pltpu.make_async_copy(h_hbm.at[pl.ds(s * HB, HB), :],
                              hsp.at[pl.ds(s * HB, HB), :], gsem1).wait()
    for k in range(nfull):
        pltpu.make_async_copy(
            srows0, acc.at[pl.ds(s * HB + k * C, C), :], ssem0).wait()
    if rem:
        pltpu.make_async_copy(
            srows0.at[pl.ds(0, rem), :],
            acc.at[pl.ds(s * HB + nfull * C, rem), :], ssem0).wait()
    plsc.subcore_barrier()

    _zero_vmem_2d(srows1, C, feat)
    pltpu.async_copy(srows0, acc.at[dstv.at[0]], ssem0, add=True)
    pltpu.async_copy(srows1, acc.at[dstv.at[0]], ssem1, add=True)

    lane_idx = [jnp.full((16,), l, jnp.int32) for l in range(16)]
    bcast_dnums = lax.GatherDimensionNumbers(
        offset_dims=(), collapsed_slice_dims=(0,), start_index_map=(0,))

    def scale(gbuf, sbuf, j):
        def group(g, _):
            wv = ewv[j, pl.ds(g * 16, 16)]
            for l in range(16):
                wl = lax.gather(wv, lane_idx[l][:, None], bcast_dnums, (1,),
                                mode=lax.GatherScatterMode.PROMISE_IN_BOUNDS)
                e = g * 16 + l
                for q in range(nq):
                    sbuf[e, pl.ds(q * 16, 16)] = (
                        gbuf[e, pl.ds(q * 16, 16)] * wl)
            return 0
        lax.fori_loop(0, C // 16, group, 0)

    def start_g(j, buf, sem):
        pltpu.async_copy(gsrc.at[srcv.at[j]], buf, sem)

    def wait_g(j, buf, sem):
        pltpu.make_async_copy(gsrc.at[srcv.at[j]], buf, sem).wait()

    def wait_s(sbuf, sem):
        pltpu.make_async_copy(sbuf, acc.at[dstv.at[0]], sem).wait()

    def step(j, gbuf, gsem, sbuf, ssem, nxt):
        wait_g(j, gbuf, gsem)
        wait_s(sbuf, ssem)
        scale(gbuf, sbuf, j)
        pltpu.async_copy(sbuf, acc.at[dstv.at[j]], ssem, add=True)
        if nxt:
            start_g(j + 2, gbuf, gsem)

    start_g(0, rows0, gsem0)
    start_g(1, rows1, gsem1)

    def pair(i, _):
        j0 = 2 * i
        step(j0, rows0, gsem0, srows0, ssem0, True)
        step(j0 + 1, rows1, gsem1, srows1, ssem1, True)
        return 0
    lax.fori_loop(0, NCHUNK // 2 - 1, pair, 0)
    step(NCHUNK - 2, rows0, gsem0, srows0, ssem0, False)
    step(NCHUNK - 1, rows1, gsem1, srows1, ssem1, False)
    wait_s(srows0, ssem0)
    wait_s(srows1, ssem1)
    plsc.subcore_barrier()

    for k in range(nfull):
        buf = srows0 if k % 2 == 0 else srows1
        pltpu.sync_copy(acc.at[pl.ds(s * HB + k * C, C), :], buf)
        pltpu.sync_copy(buf, agg_hbm.at[c, pl.ds(s * HB + k * C, C), :])
    if rem:
        pltpu.sync_copy(acc.at[pl.ds(s * HB + nfull * C, rem), :],
                        srows0.at[pl.ds(0, rem), :])
        pltpu.sync_copy(srows0.at[pl.ds(0, rem), :],
                        agg_hbm.at[c, pl.ds(s * HB + nfull * C, rem), :])


def _spmm_call(src3, dst3, ew3, h, feat, stage):
    scratch = [
        pltpu.VMEM((NCHUNK, C), jnp.int32),
        pltpu.VMEM((NCHUNK, C), jnp.int32),
        pltpu.VMEM((NCHUNK, C), jnp.float32),
        pltpu.VMEM((C, feat), jnp.float32),
        pltpu.VMEM((C, feat), jnp.float32),
        pltpu.VMEM((C, feat), jnp.float32),
        pltpu.VMEM((C, feat), jnp.float32),
    ]
    if stage:
        scratch.append(pltpu.VMEM_SHARED((N, feat), jnp.float32))
    scratch += [
        pltpu.VMEM_SHARED((N, feat), jnp.float32),
        pltpu.SemaphoreType.DMA,
        pltpu.SemaphoreType.DMA,
        pltpu.SemaphoreType.DMA,
        pltpu.SemaphoreType.DMA,
    ]
    return pl.kernel(
        functools.partial(_spmm_body, feat, stage),
        out_type=jax.ShapeDtypeStruct((NC, N, feat), jnp.float32),
        mesh=_mesh(),
        scratch_types=scratch,
        compiler_params=_SC_PARAMS,
        name=f"spmm{feat}",
    )(src3, dst3, ew3, h)


def _k1a_body(x_ref, w1_ref, h1_ref):
    h1_ref[...] = jnp.dot(x_ref[...], w1_ref[...],
                          preferred_element_type=jnp.float32)


def _k1a_call(x, W1):
    blk = 1000
    return pl.pallas_call(
        _k1a_body,
        grid=(N // blk,),
        in_specs=[
            pl.BlockSpec((blk, 128), lambda i: (i, 0)),
            pl.BlockSpec((128, 64), lambda i: (0, 0)),
        ],
        out_specs=pl.BlockSpec((blk, 64), lambda i: (i, 0)),
        out_shape=jax.ShapeDtypeStruct((N, 64), jnp.float32),
    )(x, W1)


def _k1b_body(h1_ref, degt_ref, h1p_ref, dis_ref):
    deg = degt_ref[:, 0:1] + degt_ref[:, 1:2] + 1.0
    dis = lax.rsqrt(deg)
    dis_ref[...] = dis
    h1p_ref[...] = dis * h1_ref[...]


def _k1b_call(h1, degT):
    blk = 1000
    grid = N // blk
    return pl.pallas_call(
        _k1b_body,
        grid=(grid,),
        in_specs=[
            pl.BlockSpec((blk, 64), lambda i: (i, 0)),
            pl.BlockSpec((blk, NC), lambda i: (i, 0)),
        ],
        out_specs=[
            pl.BlockSpec((blk, 64), lambda i: (i, 0)),
            pl.BlockSpec((blk, 1), lambda i: (i, 0)),
        ],
        out_shape=[
            jax.ShapeDtypeStruct((N, 64), jnp.float32),
            jax.ShapeDtypeStruct((N, 1), jnp.float32),
        ],
    )(h1, degT)


def _k2_body(a0_ref, a1_ref, h1p_ref, dis_ref, b1_ref, w2_ref, h2p_ref):
    dis = dis_ref[...]
    z = dis * (a0_ref[...] + a1_ref[...] + h1p_ref[...]) + b1_ref[...]
    z = jnp.maximum(z, 0.0)
    h2 = jnp.dot(z, w2_ref[...], preferred_element_type=jnp.float32)
    h2p_ref[...] = dis * h2


def _k2_call(a0, a1, h1p, dis, b1, W2):
    blk = 1000
    grid = N // blk
    return pl.pallas_call(
        _k2_body,
        grid=(grid,),
        in_specs=[
            pl.BlockSpec((blk, 64), lambda i: (i, 0)),
            pl.BlockSpec((blk, 64), lambda i: (i, 0)),
            pl.BlockSpec((blk, 64), lambda i: (i, 0)),
            pl.BlockSpec((blk, 1), lambda i: (i, 0)),
            pl.BlockSpec((1, 64), lambda i: (0, 0)),
            pl.BlockSpec((64, 32), lambda i: (0, 0)),
        ],
        out_specs=pl.BlockSpec((blk, 32), lambda i: (i, 0)),
        out_shape=jax.ShapeDtypeStruct((N, 32), jnp.float32),
    )(a0, a1, h1p, dis, b1, W2)


def _k3_body(a0_ref, a1_ref, h2p_ref, dis_ref, b2_ref, batch_ref, wfc_ref,
             bfc_ref, out_ref):
    dis = dis_ref[...]
    z = dis * (a0_ref[...] + a1_ref[...] + h2p_ref[...]) + b2_ref[...]
    z = jnp.maximum(z, 0.0)
    gids = lax.broadcasted_iota(jnp.int32, (G, N), 0)
    onehot = (gids == batch_ref[...]).astype(jnp.float32)
    sums = jnp.dot(onehot, z, preferred_element_type=jnp.float32)
    cnt = jnp.sum(onehot, axis=1, keepdims=True)
    pooled = sums / jnp.maximum(cnt, 1.0)
    out_ref[...] = jnp.dot(pooled, wfc_ref[...],
                           preferred_element_type=jnp.float32) + bfc_ref[...]


def _k3_call(a0, a1, h2p, dis, b2, batch2d, Wfc, bfc, flat):
    return pl.pallas_call(
        _k3_body,
        out_shape=jax.ShapeDtypeStruct((G, flat), jnp.float32),
    )(a0, a1, h2p, dis, b2, batch2d, Wfc, bfc)


def kernel(x, edge_index, edge_weight, batch, W1, b1, W2, b2, Wfc, bfc):
    flat = Wfc.shape[1]
    pad = EPAD - E
    zi = jnp.zeros((pad,), jnp.int32)
    src3 = jnp.concatenate([edge_index[0], zi]).reshape(NW, NCHUNK, C)
    dst3 = jnp.concatenate([edge_index[1], zi]).reshape(NW, NCHUNK, C)
    ew3 = jnp.concatenate(
        [edge_weight, jnp.zeros((pad,), jnp.float32)]).reshape(NW, NCHUNK, C)

    h1 = _k1a_call(x, W1)
    degp = _deg_call(dst3.reshape(NW, EPW // 128, 128),
                     ew3.reshape(NW, EPW // 128, 128))
    degT = degp.T
    h1p, dis = _k1b_call(h1, degT)

    agg1 = _spmm_call(src3, dst3, ew3, h1p, 64, True)
    h2p = _k2_call(agg1[0], agg1[1], h1p, dis, b1.reshape(1, 64), W2)

    agg2 = _spmm_call(src3, dst3, ew3, h2p, 32, True)
    return _k3_call(agg2[0, :N], agg2[1, :N], h2p, dis, b2.reshape(1, 32),
                    batch.reshape(1, N), Wfc, bfc.reshape(1, flat), flat)


# speedup vs baseline: 39.8596x; 1.0074x over previous
import functools

import jax
import jax.numpy as jnp
from jax import lax
from jax.experimental import pallas as pl
from jax.experimental.pallas import tpu as pltpu
from jax.experimental.pallas import tpu_sc as plsc

N = 10000
E = 320000
G = 16

NC = 2
NS = 16
NW = NC * NS
C = 80
NCHUNK = 125
EPW = NCHUNK * C
DC = 100
NP = 10240
STRIPE = NP // NS
WB = 128

_mesh = functools.partial(
    plsc.VectorSubcoreMesh, core_axis_name="c", subcore_axis_name="s",
    num_cores=NC, num_subcores=NS)

_SC_PARAMS = pltpu.CompilerParams(use_tc_tiling_on_sc=False)


def _zero_vmem_2d(ref, rows, feat):
    def row(i, _):
        for q in range(feat // 16):
            ref[i, pl.ds(q * 16, 16)] = jnp.zeros((16,), jnp.float32)
        return 0
    lax.fori_loop(0, rows, row, 0)


def _deg_body(dst_hbm, ew_hbm, deg_hbm, dstv, ewv, zb, acc):
    c = lax.axis_index("c")
    s = lax.axis_index("s")
    wid = s * NC + c

    pltpu.sync_copy(dst_hbm.at[wid], dstv)
    pltpu.sync_copy(ew_hbm.at[wid], ewv)

    def zrow(i, _):
        zb[pl.ds(i * 16, 16)] = jnp.zeros((16,), jnp.float32)
        return 0
    lax.fori_loop(0, STRIPE // 16, zrow, 0)
    pltpu.sync_copy(zb, acc.at[pl.ds(s * STRIPE, STRIPE)])
    plsc.subcore_barrier()

    def chunk(j, _):
        pltpu.sync_copy(ewv.at[j], acc.at[dstv.at[j]], add=True)
        return 0
    lax.fori_loop(0, EPW // DC, chunk, 0)
    plsc.subcore_barrier()

    pltpu.sync_copy(acc.at[pl.ds(s * STRIPE, STRIPE)], zb)
    pltpu.sync_copy(zb, deg_hbm.at[c, pl.ds(s * STRIPE, STRIPE)])


def _deg_call(dst2, ew2):
    return pl.kernel(
        _deg_body,
        out_type=jax.ShapeDtypeStruct((NC, NP), jnp.float32),
        mesh=_mesh(),
        scratch_types=[
            pltpu.VMEM((EPW // DC, DC), jnp.int32),
            pltpu.VMEM((EPW // DC, DC), jnp.float32),
            pltpu.VMEM((STRIPE,), jnp.float32),
            pltpu.VMEM_SHARED((NP,), jnp.float32),
        ],
        compiler_params=_SC_PARAMS,
        name="degkernel",
    )(dst2, ew2)


def _spmm_body(feat, stage, src_hbm, dst_hbm, ew_hbm, h_hbm, agg_hbm,
               srcv, dstv, ewv, rows0, rows1, srows0, srows1, *rest):
    if stage:
        hsp, acc, gsem0, gsem1, ssem0, ssem1 = rest
    else:
        acc, gsem0, gsem1, ssem0, ssem1 = rest
    c = lax.axis_index("c")
    s = lax.axis_index("s")
    wid = s * NC + c
    nq = feat // 16

    pltpu.async_copy(src_hbm.at[wid], srcv, gsem0)
    pltpu.async_copy(dst_hbm.at[wid], dstv, gsem0)
    pltpu.async_copy(ew_hbm.at[wid], ewv, gsem0)

    HB = N // NS
    if stage:
        pltpu.async_copy(h_hbm.at[pl.ds(s * HB, HB), :],
                         hsp.at[pl.ds(s * HB, HB), :], gsem1)
        gsrc = hsp
    else:
        gsrc = h_hbm

    _zero_vmem_2d(srows0, C, feat)
    nfull, rem = HB // C, HB % C
    for k in range(nfull):
        pltpu.async_copy(srows0, acc.at[pl.ds(s * HB + k * C, C), :], ssem0)
    if rem:
        pltpu.async_copy(srows0.at[pl.ds(0, rem), :],
                         acc.at[pl.ds(s * HB + nfull * C, rem), :], ssem0)

    pltpu.make_async_copy(src_hbm.at[wid], srcv, gsem0).wait()
    pltpu.make_async_copy(src_hbm.at[wid], dstv, gsem0).wait()
    pltpu.make_async_copy(ew_hbm.at[wid], ewv, gsem0).wait()
    if stage:
        pltpu.make_async_copy(h_hbm.at[pl.ds(s * HB, HB), :],
                              hsp.at[pl.ds(s * HB, HB), :], gsem1).wait()
    for k in range(nfull):
        pltpu.make_async_copy(
            srows0, acc.at[pl.ds(s * HB + k * C, C), :], ssem0).wait()
    if rem:
        pltpu.make_async_copy(
            srows0.at[pl.ds(0, rem), :],
            acc.at[pl.ds(s * HB + nfull * C, rem), :], ssem0).wait()
    plsc.subcore_barrier()

    _zero_vmem_2d(srows1, C, feat)
    pltpu.async_copy(srows0, acc.at[dstv.at[0]], ssem0, add=True)
    pltpu.async_copy(srows1, acc.at[dstv.at[0]], ssem1, add=True)

    lane_idx = [jnp.full((16,), l, jnp.int32) for l in range(16)]
    bcast_dnums = lax.GatherDimensionNumbers(
        offset_dims=(), collapsed_slice_dims=(0,), start_index_map=(0,))

    def scale(gbuf, sbuf, j):
        def group(g, _):
            wv = ewv[j, pl.ds(g * 16, 16)]
            for l in range(16):
                wl = lax.gather(wv, lane_idx[l][:, None], bcast_dnums, (1,),
                                mode=lax.GatherScatterMode.PROMISE_IN_BOUNDS)
                e = g * 16 + l
                for q in range(nq):
                    sbuf[e, pl.ds(q * 16, 16)] = (
                        gbuf[e, pl.ds(q * 16, 16)] * wl)
            return 0
        lax.fori_loop(0, C // 16, group, 0)

    def start_g(j, buf, sem):
        pltpu.async_copy(gsrc.at[srcv.at[j]], buf, sem)

    def wait_g(j, buf, sem):
        pltpu.make_async_copy(gsrc.at[srcv.at[j]], buf, sem).wait()

    def wait_s(sbuf, sem):
        pltpu.make_async_copy(sbuf, acc.at[dstv.at[0]], sem).wait()

    def step(j, gbuf, gsem, sbuf, ssem, nxt):
        wait_g(j, gbuf, gsem)
        wait_s(sbuf, ssem)
        scale(gbuf, sbuf, j)
        pltpu.async_copy(sbuf, acc.at[dstv.at[j]], ssem, add=True)
        if nxt:
            start_g(j + 2, gbuf, gsem)

    start_g(0, rows0, gsem0)
    start_g(1, rows1, gsem1)

    def pair(i, _):
        j0 = 2 * i
        step(j0, rows0, gsem0, srows0, ssem0, True)
        step(j0 + 1, rows1, gsem1, srows1, ssem1, True)
        return 0
    lax.fori_loop(0, (NCHUNK - 3) // 2, pair, 0)
    step(NCHUNK - 3, rows0, gsem0, srows0, ssem0, True)
    step(NCHUNK - 2, rows1, gsem1, srows1, ssem1, False)
    step(NCHUNK - 1, rows0, gsem0, srows0, ssem0, False)
    wait_s(srows0, ssem0)
    wait_s(srows1, ssem1)
    plsc.subcore_barrier()

    for k in range(nfull):
        buf = srows0 if k % 2 == 0 else srows1
        pltpu.sync_copy(acc.at[pl.ds(s * HB + k * C, C), :], buf)
        pltpu.sync_copy(buf, agg_hbm.at[c, pl.ds(s * HB + k * C, C), :])
    if rem:
        pltpu.sync_copy(acc.at[pl.ds(s * HB + nfull * C, rem), :],
                        srows0.at[pl.ds(0, rem), :])
        pltpu.sync_copy(srows0.at[pl.ds(0, rem), :],
                        agg_hbm.at[c, pl.ds(s * HB + nfull * C, rem), :])


def _spmm_call(src3, dst3, ew3, h, feat, stage):
    scratch = [
        pltpu.VMEM((NCHUNK, C), jnp.int32),
        pltpu.VMEM((NCHUNK, C), jnp.int32),
        pltpu.VMEM((NCHUNK, C), jnp.float32),
        pltpu.VMEM((C, feat), jnp.float32),
        pltpu.VMEM((C, feat), jnp.float32),
        pltpu.VMEM((C, feat), jnp.float32),
        pltpu.VMEM((C, feat), jnp.float32),
    ]
    if stage:
        scratch.append(pltpu.VMEM_SHARED((N, feat), jnp.float32))
    scratch += [
        pltpu.VMEM_SHARED((N, feat), jnp.float32),
        pltpu.SemaphoreType.DMA,
        pltpu.SemaphoreType.DMA,
        pltpu.SemaphoreType.DMA,
        pltpu.SemaphoreType.DMA,
    ]
    return pl.kernel(
        functools.partial(_spmm_body, feat, stage),
        out_type=jax.ShapeDtypeStruct((NC, N, feat), jnp.float32),
        mesh=_mesh(),
        scratch_types=scratch,
        compiler_params=_SC_PARAMS,
        name=f"spmm{feat}",
    )(src3, dst3, ew3, h)


def _k1a_body(x_ref, w1_ref, h1_ref):
    h1_ref[...] = jnp.dot(x_ref[...], w1_ref[...],
                          preferred_element_type=jnp.float32)


def _k1a_call(x, W1):
    blk = 1000
    return pl.pallas_call(
        _k1a_body,
        grid=(N // blk,),
        in_specs=[
            pl.BlockSpec((blk, 128), lambda i: (i, 0)),
            pl.BlockSpec((128, 64), lambda i: (0, 0)),
        ],
        out_specs=pl.BlockSpec((blk, 64), lambda i: (i, 0)),
        out_shape=jax.ShapeDtypeStruct((N, 64), jnp.float32),
    )(x, W1)


def _k1b_body(h1_ref, degt_ref, h1p_ref, dis_ref):
    deg = degt_ref[:, 0:1] + degt_ref[:, 1:2] + 1.0
    dis = lax.rsqrt(deg)
    dis_ref[...] = dis
    h1p_ref[...] = dis * h1_ref[...]


def _k1b_call(h1, degT):
    blk = 1000
    grid = N // blk
    return pl.pallas_call(
        _k1b_body,
        grid=(grid,),
        in_specs=[
            pl.BlockSpec((blk, 64), lambda i: (i, 0)),
            pl.BlockSpec((blk, NC), lambda i: (i, 0)),
        ],
        out_specs=[
            pl.BlockSpec((blk, 64), lambda i: (i, 0)),
            pl.BlockSpec((blk, 1), lambda i: (i, 0)),
        ],
        out_shape=[
            jax.ShapeDtypeStruct((N, 64), jnp.float32),
            jax.ShapeDtypeStruct((N, 1), jnp.float32),
        ],
    )(h1, degT)


def _k2_body(a0_ref, a1_ref, h1p_ref, dis_ref, b1_ref, w2_ref, h2p_ref):
    dis = dis_ref[...]
    z = dis * (a0_ref[...] + a1_ref[...] + h1p_ref[...]) + b1_ref[...]
    z = jnp.maximum(z, 0.0)
    h2 = jnp.dot(z, w2_ref[...], preferred_element_type=jnp.float32)
    h2p_ref[...] = dis * h2


def _k2_call(a0, a1, h1p, dis, b1, W2):
    blk = 1000
    grid = N // blk
    return pl.pallas_call(
        _k2_body,
        grid=(grid,),
        in_specs=[
            pl.BlockSpec((blk, 64), lambda i: (i, 0)),
            pl.BlockSpec((blk, 64), lambda i: (i, 0)),
            pl.BlockSpec((blk, 64), lambda i: (i, 0)),
            pl.BlockSpec((blk, 1), lambda i: (i, 0)),
            pl.BlockSpec((1, 64), lambda i: (0, 0)),
            pl.BlockSpec((64, 32), lambda i: (0, 0)),
        ],
        out_specs=pl.BlockSpec((blk, 32), lambda i: (i, 0)),
        out_shape=jax.ShapeDtypeStruct((N, 32), jnp.float32),
    )(a0, a1, h1p, dis, b1, W2)


def _k3_body(a0_ref, a1_ref, h2p_ref, dis_ref, b2_ref, batch_ref, wfc_ref,
             bfc_ref, out_ref):
    dis = dis_ref[...]
    z = dis * (a0_ref[...] + a1_ref[...] + h2p_ref[...]) + b2_ref[...]
    z = jnp.maximum(z, 0.0)
    gids = lax.broadcasted_iota(jnp.int32, (G, N), 0)
    onehot = (gids == batch_ref[...]).astype(jnp.float32)
    sums = jnp.dot(onehot, z, preferred_element_type=jnp.float32)
    cnt = jnp.sum(onehot, axis=1, keepdims=True)
    pooled = sums / jnp.maximum(cnt, 1.0)
    out_ref[...] = jnp.dot(pooled, wfc_ref[...],
                           preferred_element_type=jnp.float32) + bfc_ref[...]


def _k3_call(a0, a1, h2p, dis, b2, batch2d, Wfc, bfc, flat):
    return pl.pallas_call(
        _k3_body,
        out_shape=jax.ShapeDtypeStruct((G, flat), jnp.float32),
    )(a0, a1, h2p, dis, b2, batch2d, Wfc, bfc)


def kernel(x, edge_index, edge_weight, batch, W1, b1, W2, b2, Wfc, bfc):
    flat = Wfc.shape[1]
    src3 = edge_index[0].reshape(NW, NCHUNK, C)
    dst3 = edge_index[1].reshape(NW, NCHUNK, C)
    ew3 = edge_weight.reshape(NW, NCHUNK, C)

    h1 = _k1a_call(x, W1)
    degp = _deg_call(edge_index[1].reshape(NW, EPW // DC, DC),
                     edge_weight.reshape(NW, EPW // DC, DC))
    degT = degp.T
    h1p, dis = _k1b_call(h1, degT)

    agg1 = _spmm_call(src3, dst3, ew3, h1p, 64, True)
    h2p = _k2_call(agg1[0], agg1[1], h1p, dis, b1.reshape(1, 64), W2)

    agg2 = _spmm_call(src3, dst3, ew3, h2p, 32, True)
    return _k3_call(agg2[0, :N], agg2[1, :N], h2p, dis, b2.reshape(1, 32),
                    batch.reshape(1, N), Wfc, bfc.reshape(1, flat), flat)
